# baseline (speedup 1.0000x reference)
import jax
import jax.numpy as jnp
from jax import lax
from jax.experimental import pallas as pl
from jax.experimental.pallas import tpu as pltpu
from jax.experimental.pallas import tpu_sc as plsc

PAD = 1
B = 4
S = 8192
D = 1024

NC = 2
NS = 16
L = 16

WTOK = S // NS
CHUNK = 32
SLACK = 4
UNION = CHUNK + SLACK
NPAIRS = WTOK // CHUNK
SLOTS = 2
NSTEPS = NPAIRS // SLOTS


def _sc_body(ids_hbm, w_hbm, out_hbm, idsA_v, idsB_v, uidx_v, stage_v,
             tot_v, rows_v, tot_sh, gA0, gA1, sA0, sA1, sB0, sB1):
    gsemA = [gA0, gA1]
    ssemA = [sA0, sA1]
    ssemB = [sB0, sB1]
    cid = lax.axis_index("c")
    sid = lax.axis_index("s")
    tbaseA = (2 * cid) * S + sid * WTOK
    tbaseB = (2 * cid + 1) * S + sid * WTOK

    pltpu.sync_copy(ids_hbm.at[pl.ds(tbaseA, WTOK)], idsA_v)
    pltpu.sync_copy(ids_hbm.at[pl.ds(tbaseB, WTOK)], idsB_v)

    def make_cs(ref):
        def cs_body(i, carry):
            v = ref[pl.ds(i * L, L)]
            m = jnp.where(v != PAD, 1, 0).astype(jnp.int32)
            c = plsc.cumsum(m) + carry
            ref[pl.ds(i * L, L)] = c * m
            return jnp.max(c)
        return cs_body

    totalA = lax.fori_loop(0, WTOK // L, make_cs(idsA_v), jnp.int32(0))
    totalB = lax.fori_loop(0, WTOK // L, make_cs(idsB_v), jnp.int32(0))

    stage_v[...] = jnp.full((L,), totalA, jnp.int32)
    pltpu.sync_copy(stage_v, tot_sh.at[pl.ds(sid * L, L)])
    stage_v[...] = jnp.full((L,), totalB, jnp.int32)
    pltpu.sync_copy(stage_v, tot_sh.at[pl.ds((NS + sid) * L, L)])
    plsc.subcore_barrier()
    pltpu.sync_copy(tot_sh, tot_v)

    offA = jnp.int32(0)
    offB = jnp.int32(0)
    for j in range(NS):
        tA = jnp.max(tot_v[pl.ds(j * L, L)])
        tB = jnp.max(tot_v[pl.ds((NS + j) * L, L)])
        keep = j < sid
        offA = offA + jnp.where(keep, tA, 0).astype(jnp.int32)
        offB = offB + jnp.where(keep, tB, 0).astype(jnp.int32)

    def make_idx(ref, off):
        def idx_body(i, _):
            e = ref[pl.ds(i * L, L)]
            m = jnp.where(e > 0, 1, 0).astype(jnp.int32)
            ref[pl.ds(i * L, L)] = e + off * m + PAD
            return 0
        return idx_body

    lax.fori_loop(0, WTOK // L, make_idx(idsA_v, offA), 0)
    lax.fori_loop(0, WTOK // L, make_idx(idsB_v, offB), 0)

    lane = lax.broadcasted_iota(jnp.int32, (L,), 0)

    def flags(k):
        vA = idsA_v[pl.ds(k * CHUNK, L)]
        wA = idsA_v[pl.ds(k * CHUNK + CHUNK - L, L)]
        vB = idsB_v[pl.ds(k * CHUNK, L)]
        wB = idsB_v[pl.ds(k * CHUNK + CHUNK - L, L)]
        fA, lA = vA[0], wA[L - 1]
        fB, lB = vB[0], wB[L - 1]
        cleanA = jnp.logical_and(fA > 1, lA - fA == CHUNK - 1)
        cleanB = jnp.logical_and(fB > 1, lB - fB == CHUNK - 1)
        both = jnp.logical_and(cleanA, cleanB)
        fmin = jnp.minimum(fA, fB)
        dA = fA - fmin
        dB = fB - fmin
        dmax = jnp.maximum(dA, dB)
        eq = jnp.logical_and(both, dmax == 0)
        un = jnp.logical_and(jnp.logical_and(both, dmax > 0), dmax <= SLACK)
        sp = jnp.logical_not(jnp.logical_or(eq, un))
        return eq, un, sp, fmin, dA, dB

    def g_eq(k, b):
        pltpu.async_copy(
            w_hbm.at[idsA_v.at[pl.ds(k * CHUNK, CHUNK)]],
            rows_v.at[b, pl.ds(0, CHUNK)], gsemA[b])

    def g_eq_wait(b):
        pltpu.make_async_copy(
            w_hbm.at[idsA_v.at[pl.ds(0, CHUNK)]],
            rows_v.at[b, pl.ds(0, CHUNK)], gsemA[b]).wait()

    def g_un(fmin, b):
        base = b * ((UNION + L - 1) // L * L)
        for g in range((UNION + L - 1) // L):
            uidx_v[pl.ds(base + g * L, L)] = fmin + g * L + lane
        pltpu.async_copy(
            w_hbm.at[uidx_v.at[pl.ds(base, UNION)]],
            rows_v.at[b, pl.ds(0, UNION)], gsemA[b])

    def g_un_wait(b):
        pltpu.make_async_copy(
            w_hbm.at[uidx_v.at[pl.ds(0, UNION)]],
            rows_v.at[b, pl.ds(0, UNION)], gsemA[b]).wait()

    def g_row(idx_ref, k, b):
        pltpu.async_copy(
            w_hbm.at[idx_ref.at[pl.ds(k * CHUNK, CHUNK)]],
            rows_v.at[b, pl.ds(0, CHUNK)], gsemA[b])

    def scat(src_off, b, dst, sem):
        pltpu.async_copy(
            rows_v.at[b, pl.ds(src_off, CHUNK)],
            out_hbm.at[pl.ds(dst, CHUNK)], sem)

    def scat_wait(b, sem):
        pltpu.make_async_copy(
            rows_v.at[b, pl.ds(0, CHUNK)], out_hbm.at[pl.ds(0, CHUNK)],
            sem).wait()

    def issue(k, b):
        eq, un, sp, fmin, dA, dB = flags(k)

        @pl.when(eq)
        def _():
            g_eq(k, b)

        @pl.when(un)
        def _():
            g_un(fmin, b)

    def consume(k, b):
        eq, un, sp, fmin, dA, dB = flags(k)
        dstA = tbaseA + k * CHUNK
        dstB = tbaseB + k * CHUNK

        @pl.when(eq)
        def _():
            g_eq_wait(b)
            scat(0, b, dstA, ssemA[b])
            scat(0, b, dstB, ssemB[b])
            scat_wait(b, ssemA[b])
            scat_wait(b, ssemB[b])

        @pl.when(un)
        def _():
            g_un_wait(b)
            scat(dA, b, dstA, ssemA[b])
            scat(dB, b, dstB, ssemB[b])
            scat_wait(b, ssemA[b])
            scat_wait(b, ssemB[b])

        @pl.when(sp)
        def _():
            g_row(idsA_v, k, b)
            g_eq_wait(b)
            scat(0, b, dstA, ssemA[b])
            scat_wait(b, ssemA[b])
            g_row(idsB_v, k, b)
            g_eq_wait(b)
            scat(0, b, dstB, ssemB[b])
            scat_wait(b, ssemB[b])

    for b in range(SLOTS):
        issue(b, b)

    def pipe_body(step, _):
        for b in range(SLOTS):
            k = step * SLOTS + b
            consume(k, b)
            issue(k + SLOTS, b)
        return 0

    lax.fori_loop(0, NSTEPS - 1, pipe_body, 0)

    for b in range(SLOTS):
        consume((NSTEPS - 1) * SLOTS + b, b)


@jax.jit
def _sc_embed(ids_flat, weights):
    mesh = plsc.VectorSubcoreMesh(
        core_axis_name="c", subcore_axis_name="s",
        num_cores=NC, num_subcores=NS)
    uidx_words = SLOTS * ((UNION + L - 1) // L * L)
    f = pl.kernel(
        _sc_body,
        out_type=jax.ShapeDtypeStruct((B * S, 8, D // 8), jnp.float32),
        mesh=mesh,
        compiler_params=pltpu.CompilerParams(needs_layout_passes=False),
        scratch_types=[
            pltpu.VMEM((WTOK,), jnp.int32),
            pltpu.VMEM((WTOK,), jnp.int32),
            pltpu.VMEM((uidx_words,), jnp.int32),
            pltpu.VMEM((L,), jnp.int32),
            pltpu.VMEM((2 * NS * L,), jnp.int32),
            pltpu.VMEM((SLOTS, UNION, 8, D // 8), jnp.float32),
            pltpu.VMEM_SHARED((2 * NS * L,), jnp.int32),
            pltpu.SemaphoreType.DMA,
            pltpu.SemaphoreType.DMA,
            pltpu.SemaphoreType.DMA,
            pltpu.SemaphoreType.DMA,
            pltpu.SemaphoreType.DMA,
            pltpu.SemaphoreType.DMA,
        ],
    )
    return f(ids_flat, weights.reshape(-1, 8, D // 8))


def kernel(input_ids, weights):
    out = _sc_embed(input_ids.reshape(-1), weights)
    return out.reshape(B, S, D)


# speedup vs baseline: 1.0591x; 1.0591x over previous
import jax
import jax.numpy as jnp
from jax import lax
from jax.experimental import pallas as pl
from jax.experimental.pallas import tpu as pltpu
from jax.experimental.pallas import tpu_sc as plsc

PAD = 1
B = 4
S = 8192
D = 1024

NC = 2
NS = 16
L = 16

WTOK = S // NS
CHUNK = 32
SLACK = 4
UNION = CHUNK + SLACK
NPAIRS = WTOK // CHUNK
SLOTS = 2
NSTEPS = NPAIRS // SLOTS


def _sc_body(ids_hbm, w_hbm, out_hbm, idsA_v, idsB_v, uidx_v, stage_v,
             tot_v, rows_v, tot_sh, gA0, gA1, sA0, sA1, sB0, sB1):
    gsemA = [gA0, gA1]
    ssemA = [sA0, sA1]
    ssemB = [sB0, sB1]
    cid = lax.axis_index("c")
    sid = lax.axis_index("s")
    tbaseA = (2 * cid) * S + sid * WTOK
    tbaseB = (2 * cid + 1) * S + sid * WTOK

    pltpu.sync_copy(ids_hbm.at[pl.ds(tbaseA, WTOK)], idsA_v)
    pltpu.sync_copy(ids_hbm.at[pl.ds(tbaseB, WTOK)], idsB_v)

    def make_cs(ref):
        def cs_body(i, carry):
            v = ref[pl.ds(i * L, L)]
            m = jnp.where(v != PAD, 1, 0).astype(jnp.int32)
            c = plsc.cumsum(m) + carry
            ref[pl.ds(i * L, L)] = c * m
            return jnp.max(c)
        return cs_body

    totalA = lax.fori_loop(0, WTOK // L, make_cs(idsA_v), jnp.int32(0))
    totalB = lax.fori_loop(0, WTOK // L, make_cs(idsB_v), jnp.int32(0))

    stage_v[...] = jnp.full((L,), totalA, jnp.int32)
    pltpu.sync_copy(stage_v, tot_sh.at[pl.ds(sid * L, L)])
    stage_v[...] = jnp.full((L,), totalB, jnp.int32)
    pltpu.sync_copy(stage_v, tot_sh.at[pl.ds((NS + sid) * L, L)])
    plsc.subcore_barrier()
    pltpu.sync_copy(tot_sh, tot_v)

    offA = jnp.int32(0)
    offB = jnp.int32(0)
    for j in range(NS):
        tA = jnp.max(tot_v[pl.ds(j * L, L)])
        tB = jnp.max(tot_v[pl.ds((NS + j) * L, L)])
        keep = j < sid
        offA = offA + jnp.where(keep, tA, 0).astype(jnp.int32)
        offB = offB + jnp.where(keep, tB, 0).astype(jnp.int32)

    def make_idx(ref, off):
        def idx_body(i, _):
            e = ref[pl.ds(i * L, L)]
            m = jnp.where(e > 0, 1, 0).astype(jnp.int32)
            ref[pl.ds(i * L, L)] = e + off * m + PAD
            return 0
        return idx_body

    lax.fori_loop(0, WTOK // L, make_idx(idsA_v, offA), 0)
    lax.fori_loop(0, WTOK // L, make_idx(idsB_v, offB), 0)

    lane = lax.broadcasted_iota(jnp.int32, (L,), 0)

    def flags(k):
        vA = idsA_v[pl.ds(k * CHUNK, L)]
        wA = idsA_v[pl.ds(k * CHUNK + CHUNK - L, L)]
        vB = idsB_v[pl.ds(k * CHUNK, L)]
        wB = idsB_v[pl.ds(k * CHUNK + CHUNK - L, L)]
        fA, lA = vA[0], wA[L - 1]
        fB, lB = vB[0], wB[L - 1]
        cleanA = jnp.logical_and(fA > 1, lA - fA == CHUNK - 1)
        cleanB = jnp.logical_and(fB > 1, lB - fB == CHUNK - 1)
        both = jnp.logical_and(cleanA, cleanB)
        fmin = jnp.minimum(fA, fB)
        dA = fA - fmin
        dB = fB - fmin
        dmax = jnp.maximum(dA, dB)
        eq = jnp.logical_and(both, dmax == 0)
        un = jnp.logical_and(jnp.logical_and(both, dmax > 0), dmax <= SLACK)
        sp = jnp.logical_not(jnp.logical_or(eq, un))
        return eq, un, sp, fmin, dA, dB

    def g_eq(k, b):
        pltpu.async_copy(
            w_hbm.at[idsA_v.at[pl.ds(k * CHUNK, CHUNK)]],
            rows_v.at[b, pl.ds(0, CHUNK)], gsemA[b])

    def g_eq_wait(b):
        pltpu.make_async_copy(
            w_hbm.at[idsA_v.at[pl.ds(0, CHUNK)]],
            rows_v.at[b, pl.ds(0, CHUNK)], gsemA[b]).wait()

    def g_un(fmin, b):
        base = b * ((UNION + L - 1) // L * L)
        for g in range((UNION + L - 1) // L):
            uidx_v[pl.ds(base + g * L, L)] = fmin + g * L + lane
        pltpu.async_copy(
            w_hbm.at[uidx_v.at[pl.ds(base, UNION)]],
            rows_v.at[b, pl.ds(0, UNION)], gsemA[b])

    def g_un_wait(b):
        pltpu.make_async_copy(
            w_hbm.at[uidx_v.at[pl.ds(0, UNION)]],
            rows_v.at[b, pl.ds(0, UNION)], gsemA[b]).wait()

    def g_row(idx_ref, k, b):
        pltpu.async_copy(
            w_hbm.at[idx_ref.at[pl.ds(k * CHUNK, CHUNK)]],
            rows_v.at[b, pl.ds(0, CHUNK)], gsemA[b])

    def scat(src_off, b, dst, sem):
        pltpu.async_copy(
            rows_v.at[b, pl.ds(src_off, CHUNK)],
            out_hbm.at[pl.ds(dst, CHUNK)], sem)

    def scat_wait(b, sem):
        pltpu.make_async_copy(
            rows_v.at[b, pl.ds(0, CHUNK)], out_hbm.at[pl.ds(0, CHUNK)],
            sem).wait()

    def issue(k, b):
        eq, un, sp, fmin, dA, dB = flags(k)

        @pl.when(eq)
        def _():
            g_eq(k, b)

        @pl.when(un)
        def _():
            g_un(fmin, b)

    def consume(k, b):
        eq, un, sp, fmin, dA, dB = flags(k)
        dstA = tbaseA + k * CHUNK
        dstB = tbaseB + k * CHUNK

        @pl.when(eq)
        def _():
            g_eq_wait(b)
            scat(0, b, dstA, ssemA[b])
            scat(0, b, dstB, ssemB[b])
            scat_wait(b, ssemA[b])
            scat_wait(b, ssemB[b])

        @pl.when(un)
        def _():
            g_un_wait(b)
            scat(dA, b, dstA, ssemA[b])
            scat(dB, b, dstB, ssemB[b])
            scat_wait(b, ssemA[b])
            scat_wait(b, ssemB[b])

        @pl.when(sp)
        def _():
            g_row(idsA_v, k, b)
            g_eq_wait(b)
            scat(0, b, dstA, ssemA[b])
            scat_wait(b, ssemA[b])
            g_row(idsB_v, k, b)
            g_eq_wait(b)
            scat(0, b, dstB, ssemB[b])
            scat_wait(b, ssemB[b])

    for b in range(SLOTS):
        issue(b, b)

    def pipe_body(step, _):
        for b in range(SLOTS):
            k = step * SLOTS + b
            consume(k, b)
            issue(k + SLOTS, b)
        return 0

    lax.fori_loop(0, NSTEPS - 1, pipe_body, 0)

    for b in range(SLOTS):
        consume((NSTEPS - 1) * SLOTS + b, b)


@jax.jit
def _sc_embed(ids_flat, weights):
    mesh = plsc.VectorSubcoreMesh(
        core_axis_name="c", subcore_axis_name="s",
        num_cores=NC, num_subcores=NS)
    uidx_words = SLOTS * ((UNION + L - 1) // L * L)
    f = pl.kernel(
        _sc_body,
        out_type=jax.ShapeDtypeStruct((B * S, D), jnp.float32),
        mesh=mesh,
        compiler_params=pltpu.CompilerParams(
            needs_layout_passes=False, use_tc_tiling_on_sc=False),
        scratch_types=[
            pltpu.VMEM((WTOK,), jnp.int32),
            pltpu.VMEM((WTOK,), jnp.int32),
            pltpu.VMEM((uidx_words,), jnp.int32),
            pltpu.VMEM((L,), jnp.int32),
            pltpu.VMEM((2 * NS * L,), jnp.int32),
            pltpu.VMEM((SLOTS, UNION, D), jnp.float32),
            pltpu.VMEM_SHARED((2 * NS * L,), jnp.int32),
            pltpu.SemaphoreType.DMA,
            pltpu.SemaphoreType.DMA,
            pltpu.SemaphoreType.DMA,
            pltpu.SemaphoreType.DMA,
            pltpu.SemaphoreType.DMA,
            pltpu.SemaphoreType.DMA,
        ],
    )
    return f(ids_flat, weights)


def kernel(input_ids, weights):
    out = _sc_embed(input_ids.reshape(-1), weights)
    return out.reshape(B, S, D)


# speedup vs baseline: 2.6250x; 2.4785x over previous
import jax
import jax.numpy as jnp
from jax import lax
from jax.experimental import pallas as pl
from jax.experimental.pallas import tpu as pltpu
from jax.experimental.pallas import tpu_sc as plsc

PAD = 1
B = 4
S = 8192
D = 1024

NC = 2
NS = 16
L = 16

WTOK = S // NS
CHUNK = 32
SLACK = 4
UNION = CHUNK + SLACK
NPAIRS = WTOK // CHUNK
SLOTS = 2
NSTEPS = NPAIRS // SLOTS
UPAD = (UNION + L - 1) // L * L


def _sc_body(ids_hbm, w_hbm, out_hbm, idsA_v, idsB_v, uidx_v, sidx_v,
             stage_v, tot_v, rows_v, tot_sh, gA0, gA1, sA0, sA1, sB0, sB1):
    gsemA = [gA0, gA1]
    ssemA = [sA0, sA1]
    ssemB = [sB0, sB1]
    cid = lax.axis_index("c")
    sid = lax.axis_index("s")
    tbaseA = (2 * cid) * S + sid * WTOK
    tbaseB = (2 * cid + 1) * S + sid * WTOK

    pltpu.sync_copy(ids_hbm.at[pl.ds(tbaseA, WTOK)], idsA_v)
    pltpu.sync_copy(ids_hbm.at[pl.ds(tbaseB, WTOK)], idsB_v)

    def make_cs(ref):
        def cs_body(i, carry):
            v = ref[pl.ds(i * L, L)]
            m = jnp.where(v != PAD, 1, 0).astype(jnp.int32)
            c = plsc.cumsum(m) + carry
            ref[pl.ds(i * L, L)] = c * m
            return jnp.max(c)
        return cs_body

    totalA = lax.fori_loop(0, WTOK // L, make_cs(idsA_v), jnp.int32(0))
    totalB = lax.fori_loop(0, WTOK // L, make_cs(idsB_v), jnp.int32(0))

    stage_v[...] = jnp.full((L,), totalA, jnp.int32)
    pltpu.sync_copy(stage_v, tot_sh.at[pl.ds(sid * L, L)])
    stage_v[...] = jnp.full((L,), totalB, jnp.int32)
    pltpu.sync_copy(stage_v, tot_sh.at[pl.ds((NS + sid) * L, L)])
    plsc.subcore_barrier()
    pltpu.sync_copy(tot_sh, tot_v)

    offA = jnp.int32(0)
    offB = jnp.int32(0)
    for j in range(NS):
        tA = jnp.max(tot_v[pl.ds(j * L, L)])
        tB = jnp.max(tot_v[pl.ds((NS + j) * L, L)])
        keep = j < sid
        offA = offA + jnp.where(keep, tA, 0).astype(jnp.int32)
        offB = offB + jnp.where(keep, tB, 0).astype(jnp.int32)

    def make_idx(ref, off):
        def idx_body(i, _):
            e = ref[pl.ds(i * L, L)]
            m = jnp.where(e > 0, 1, 0).astype(jnp.int32)
            ref[pl.ds(i * L, L)] = e + off * m + PAD
            return 0
        return idx_body

    lax.fori_loop(0, WTOK // L, make_idx(idsA_v, offA), 0)
    lax.fori_loop(0, WTOK // L, make_idx(idsB_v, offB), 0)

    lane = lax.broadcasted_iota(jnp.int32, (L,), 0)

    def run_continues(ref, f, d, k):
        lo = jnp.maximum(k * CHUNK - L, 0)
        vprev = ref[pl.ds(lo, L)]
        mism_lo = jnp.sum(jnp.where(
            jnp.logical_and(lane >= L - d, vprev != f - L + lane), 1, 0))
        hi = jnp.minimum(k * CHUNK + CHUNK, WTOK - L)
        vnext = ref[pl.ds(hi, L)]
        mism_hi = jnp.sum(jnp.where(
            jnp.logical_and(lane < SLACK - d, vnext != f + CHUNK + lane),
            1, 0))
        return (mism_lo + mism_hi) == 0

    def flags(k):
        vA = idsA_v[pl.ds(k * CHUNK, L)]
        wA = idsA_v[pl.ds(k * CHUNK + CHUNK - L, L)]
        vB = idsB_v[pl.ds(k * CHUNK, L)]
        wB = idsB_v[pl.ds(k * CHUNK + CHUNK - L, L)]
        fA, lA = vA[0], wA[L - 1]
        fB, lB = vB[0], wB[L - 1]
        cleanA = jnp.logical_and(fA > 1, lA - fA == CHUNK - 1)
        cleanB = jnp.logical_and(fB > 1, lB - fB == CHUNK - 1)
        both = jnp.logical_and(cleanA, cleanB)
        fmin = jnp.minimum(fA, fB)
        dA = fA - fmin
        dB = fB - fmin
        dmax = jnp.maximum(dA, dB)
        eq = jnp.logical_and(both, dmax == 0)
        un = jnp.logical_and(jnp.logical_and(both, dmax > 0),
                             dmax <= SLACK)
        ok = jnp.where(dA > 0, run_continues(idsA_v, fA, dA, k),
                       run_continues(idsB_v, fB, dB, k))
        inner = jnp.logical_and(k >= 1, k < NPAIRS - 1)
        un = jnp.logical_and(un, jnp.logical_and(ok, inner))
        sp = jnp.logical_not(jnp.logical_or(eq, un))
        return eq, un, sp, fmin, dA, dB

    def g_eq(k, b):
        pltpu.async_copy(
            w_hbm.at[idsA_v.at[pl.ds(k * CHUNK, CHUNK)]],
            rows_v.at[b, pl.ds(0, CHUNK)], gsemA[b])

    def g_eq_wait(b):
        pltpu.make_async_copy(
            w_hbm.at[idsA_v.at[pl.ds(0, CHUNK)]],
            rows_v.at[b, pl.ds(0, CHUNK)], gsemA[b]).wait()

    def g_un(fmin, b):
        for g in range(UPAD // L):
            uidx_v[pl.ds(b * UPAD + g * L, L)] = fmin + g * L + lane
        pltpu.async_copy(
            w_hbm.at[uidx_v.at[pl.ds(b * UPAD, UNION)]],
            rows_v.at[b, pl.ds(0, UNION)], gsemA[b])

    def g_un_wait(b):
        pltpu.make_async_copy(
            w_hbm.at[uidx_v.at[pl.ds(0, UNION)]],
            rows_v.at[b, pl.ds(0, UNION)], gsemA[b]).wait()

    def g_row(idx_ref, k, b):
        pltpu.async_copy(
            w_hbm.at[idx_ref.at[pl.ds(k * CHUNK, CHUNK)]],
            rows_v.at[b, pl.ds(0, CHUNK)], gsemA[b])

    def scat(b, dst, sem):
        pltpu.async_copy(
            rows_v.at[b, pl.ds(0, CHUNK)],
            out_hbm.at[pl.ds(dst, CHUNK)], sem)

    def scat_wait(b, sem):
        pltpu.make_async_copy(
            rows_v.at[b, pl.ds(0, CHUNK)], out_hbm.at[pl.ds(0, CHUNK)],
            sem).wait()

    def scat36(b, t0, sem):
        sidx_v[b, pl.ds(0, L)] = t0 + lane
        sidx_v[b, pl.ds(L, L)] = t0 + L + lane
        sidx_v[b, pl.ds(UNION - L, L)] = t0 + UNION - L + lane
        pltpu.async_copy(
            rows_v.at[b, pl.ds(0, UNION)], out_hbm.at[sidx_v.at[b]], sem)

    def scat36_wait(b, sem):
        pltpu.make_async_copy(
            rows_v.at[b, pl.ds(0, UNION)], out_hbm.at[sidx_v.at[b]],
            sem).wait()

    def issue(k, b):
        eq, un, sp, fmin, dA, dB = flags(k)

        @pl.when(eq)
        def _():
            g_eq(k, b)

        @pl.when(un)
        def _():
            g_un(fmin, b)

    def consume(k, b):
        eq, un, sp, fmin, dA, dB = flags(k)
        dstA = tbaseA + k * CHUNK
        dstB = tbaseB + k * CHUNK

        @pl.when(eq)
        def _():
            g_eq_wait(b)
            scat(b, dstA, ssemA[b])
            scat(b, dstB, ssemB[b])
            scat_wait(b, ssemA[b])
            scat_wait(b, ssemB[b])

        @pl.when(un)
        def _():
            g_un_wait(b)

            @pl.when(dA > 0)
            def _():
                scat36(b, dstA - dA, ssemA[b])
                scat(b, dstB, ssemB[b])
                scat36_wait(b, ssemA[b])
                scat_wait(b, ssemB[b])

            @pl.when(dA == 0)
            def _():
                scat(b, dstA, ssemA[b])
                scat36(b, dstB - dB, ssemB[b])
                scat_wait(b, ssemA[b])
                scat36_wait(b, ssemB[b])

        @pl.when(sp)
        def _():
            g_row(idsA_v, k, b)
            g_eq_wait(b)
            scat(b, dstA, ssemA[b])
            scat_wait(b, ssemA[b])
            g_row(idsB_v, k, b)
            g_eq_wait(b)
            scat(b, dstB, ssemB[b])
            scat_wait(b, ssemB[b])

    for b in range(SLOTS):
        issue(b, b)

    def pipe_body(step, _):
        for b in range(SLOTS):
            k = step * SLOTS + b
            consume(k, b)
            issue(k + SLOTS, b)
        return 0

    lax.fori_loop(0, NSTEPS - 1, pipe_body, 0)

    for b in range(SLOTS):
        consume((NSTEPS - 1) * SLOTS + b, b)


@jax.jit
def _sc_embed(ids_flat, weights):
    mesh = plsc.VectorSubcoreMesh(
        core_axis_name="c", subcore_axis_name="s",
        num_cores=NC, num_subcores=NS)
    f = pl.kernel(
        _sc_body,
        out_type=jax.ShapeDtypeStruct((B * S, D), jnp.float32),
        mesh=mesh,
        compiler_params=pltpu.CompilerParams(needs_layout_passes=False),
        scratch_types=[
            pltpu.VMEM((WTOK,), jnp.int32),
            pltpu.VMEM((WTOK,), jnp.int32),
            pltpu.VMEM((SLOTS * UPAD,), jnp.int32),
            pltpu.VMEM((SLOTS, UNION), jnp.int32),
            pltpu.VMEM((L,), jnp.int32),
            pltpu.VMEM((2 * NS * L,), jnp.int32),
            pltpu.VMEM((SLOTS, UNION, D), jnp.float32),
            pltpu.VMEM_SHARED((2 * NS * L,), jnp.int32),
            pltpu.SemaphoreType.DMA,
            pltpu.SemaphoreType.DMA,
            pltpu.SemaphoreType.DMA,
            pltpu.SemaphoreType.DMA,
            pltpu.SemaphoreType.DMA,
            pltpu.SemaphoreType.DMA,
        ],
    )
    return f(ids_flat, weights)


def kernel(input_ids, weights):
    out = _sc_embed(input_ids.reshape(-1), weights)
    return out.reshape(B, S, D)


# speedup vs baseline: 2.6421x; 1.0065x over previous
import jax
import jax.numpy as jnp
from jax import lax
from jax.experimental import pallas as pl
from jax.experimental.pallas import tpu as pltpu
from jax.experimental.pallas import tpu_sc as plsc

PAD = 1
B = 4
S = 8192
D = 1024

NC = 2
NS = 16
L = 16

WTOK = S // NS
CHUNK = 64
SLACK = 4
UNION = CHUNK + SLACK
NPAIRS = WTOK // CHUNK
SLOTS = 1
NSTEPS = NPAIRS // SLOTS
UPAD = (UNION + L - 1) // L * L


def _sc_body(ids_hbm, w_hbm, out_hbm, idsA_v, idsB_v, uidx_v, sidx_v,
             stage_v, tot_v, rows_v, tot_sh, gA0, sA0, sB0):
    gsemA = [gA0]
    ssemA = [sA0]
    ssemB = [sB0]
    cid = lax.axis_index("c")
    sid = lax.axis_index("s")
    tbaseA = (2 * cid) * S + sid * WTOK
    tbaseB = (2 * cid + 1) * S + sid * WTOK

    pltpu.sync_copy(ids_hbm.at[pl.ds(tbaseA, WTOK)], idsA_v)
    pltpu.sync_copy(ids_hbm.at[pl.ds(tbaseB, WTOK)], idsB_v)

    def make_cs(ref):
        def cs_body(i, carry):
            v = ref[pl.ds(i * L, L)]
            m = jnp.where(v != PAD, 1, 0).astype(jnp.int32)
            c = plsc.cumsum(m) + carry
            ref[pl.ds(i * L, L)] = c * m
            return jnp.max(c)
        return cs_body

    totalA = lax.fori_loop(0, WTOK // L, make_cs(idsA_v), jnp.int32(0))
    totalB = lax.fori_loop(0, WTOK // L, make_cs(idsB_v), jnp.int32(0))

    stage_v[...] = jnp.full((L,), totalA, jnp.int32)
    pltpu.sync_copy(stage_v, tot_sh.at[pl.ds(sid * L, L)])
    stage_v[...] = jnp.full((L,), totalB, jnp.int32)
    pltpu.sync_copy(stage_v, tot_sh.at[pl.ds((NS + sid) * L, L)])
    plsc.subcore_barrier()
    pltpu.sync_copy(tot_sh, tot_v)

    offA = jnp.int32(0)
    offB = jnp.int32(0)
    for j in range(NS):
        tA = jnp.max(tot_v[pl.ds(j * L, L)])
        tB = jnp.max(tot_v[pl.ds((NS + j) * L, L)])
        keep = j < sid
        offA = offA + jnp.where(keep, tA, 0).astype(jnp.int32)
        offB = offB + jnp.where(keep, tB, 0).astype(jnp.int32)

    def make_idx(ref, off):
        def idx_body(i, _):
            e = ref[pl.ds(i * L, L)]
            m = jnp.where(e > 0, 1, 0).astype(jnp.int32)
            ref[pl.ds(i * L, L)] = e + off * m + PAD
            return 0
        return idx_body

    lax.fori_loop(0, WTOK // L, make_idx(idsA_v, offA), 0)
    lax.fori_loop(0, WTOK // L, make_idx(idsB_v, offB), 0)

    lane = lax.broadcasted_iota(jnp.int32, (L,), 0)

    def run_continues(ref, f, d, k):
        lo = jnp.maximum(k * CHUNK - L, 0)
        vprev = ref[pl.ds(lo, L)]
        mism_lo = jnp.sum(jnp.where(
            jnp.logical_and(lane >= L - d, vprev != f - L + lane), 1, 0))
        hi = jnp.minimum(k * CHUNK + CHUNK, WTOK - L)
        vnext = ref[pl.ds(hi, L)]
        mism_hi = jnp.sum(jnp.where(
            jnp.logical_and(lane < SLACK - d, vnext != f + CHUNK + lane),
            1, 0))
        return (mism_lo + mism_hi) == 0

    def flags(k):
        vA = idsA_v[pl.ds(k * CHUNK, L)]
        wA = idsA_v[pl.ds(k * CHUNK + CHUNK - L, L)]
        vB = idsB_v[pl.ds(k * CHUNK, L)]
        wB = idsB_v[pl.ds(k * CHUNK + CHUNK - L, L)]
        fA, lA = vA[0], wA[L - 1]
        fB, lB = vB[0], wB[L - 1]
        cleanA = jnp.logical_and(fA > 1, lA - fA == CHUNK - 1)
        cleanB = jnp.logical_and(fB > 1, lB - fB == CHUNK - 1)
        both = jnp.logical_and(cleanA, cleanB)
        fmin = jnp.minimum(fA, fB)
        dA = fA - fmin
        dB = fB - fmin
        dmax = jnp.maximum(dA, dB)
        eq = jnp.logical_and(both, dmax == 0)
        un = jnp.logical_and(jnp.logical_and(both, dmax > 0),
                             dmax <= SLACK)
        ok = jnp.where(dA > 0, run_continues(idsA_v, fA, dA, k),
                       run_continues(idsB_v, fB, dB, k))
        inner = jnp.logical_and(k >= 1, k < NPAIRS - 1)
        un = jnp.logical_and(un, jnp.logical_and(ok, inner))
        sp = jnp.logical_not(jnp.logical_or(eq, un))
        return eq, un, sp, fmin, dA, dB

    def g_eq(k, b):
        pltpu.async_copy(
            w_hbm.at[idsA_v.at[pl.ds(k * CHUNK, CHUNK)]],
            rows_v.at[b, pl.ds(0, CHUNK)], gsemA[b])

    def g_eq_wait(b):
        pltpu.make_async_copy(
            w_hbm.at[idsA_v.at[pl.ds(0, CHUNK)]],
            rows_v.at[b, pl.ds(0, CHUNK)], gsemA[b]).wait()

    def g_un(fmin, b):
        for g in range(UPAD // L):
            uidx_v[pl.ds(b * UPAD + g * L, L)] = fmin + g * L + lane
        pltpu.async_copy(
            w_hbm.at[uidx_v.at[pl.ds(b * UPAD, UNION)]],
            rows_v.at[b, pl.ds(0, UNION)], gsemA[b])

    def g_un_wait(b):
        pltpu.make_async_copy(
            w_hbm.at[uidx_v.at[pl.ds(0, UNION)]],
            rows_v.at[b, pl.ds(0, UNION)], gsemA[b]).wait()

    def g_row(idx_ref, k, b):
        pltpu.async_copy(
            w_hbm.at[idx_ref.at[pl.ds(k * CHUNK, CHUNK)]],
            rows_v.at[b, pl.ds(0, CHUNK)], gsemA[b])

    def scat(b, dst, sem):
        pltpu.async_copy(
            rows_v.at[b, pl.ds(0, CHUNK)],
            out_hbm.at[pl.ds(dst, CHUNK)], sem)

    def scat_wait(b, sem):
        pltpu.make_async_copy(
            rows_v.at[b, pl.ds(0, CHUNK)], out_hbm.at[pl.ds(0, CHUNK)],
            sem).wait()

    def scat36(b, t0, sem):
        for g in range(UNION // L):
            sidx_v[b, pl.ds(g * L, L)] = t0 + g * L + lane
        sidx_v[b, pl.ds(UNION - L, L)] = t0 + UNION - L + lane
        pltpu.async_copy(
            rows_v.at[b, pl.ds(0, UNION)], out_hbm.at[sidx_v.at[b]], sem)

    def scat36_wait(b, sem):
        pltpu.make_async_copy(
            rows_v.at[b, pl.ds(0, UNION)], out_hbm.at[sidx_v.at[b]],
            sem).wait()

    def issue(k, b):
        eq, un, sp, fmin, dA, dB = flags(k)

        @pl.when(eq)
        def _():
            g_eq(k, b)

        @pl.when(un)
        def _():
            g_un(fmin, b)

    def consume(k, b):
        eq, un, sp, fmin, dA, dB = flags(k)
        dstA = tbaseA + k * CHUNK
        dstB = tbaseB + k * CHUNK

        @pl.when(eq)
        def _():
            g_eq_wait(b)
            scat(b, dstA, ssemA[b])
            scat(b, dstB, ssemB[b])
            scat_wait(b, ssemA[b])
            scat_wait(b, ssemB[b])

        @pl.when(un)
        def _():
            g_un_wait(b)

            @pl.when(dA > 0)
            def _():
                scat36(b, dstA - dA, ssemA[b])
                scat(b, dstB, ssemB[b])
                scat36_wait(b, ssemA[b])
                scat_wait(b, ssemB[b])

            @pl.when(dA == 0)
            def _():
                scat(b, dstA, ssemA[b])
                scat36(b, dstB - dB, ssemB[b])
                scat_wait(b, ssemA[b])
                scat36_wait(b, ssemB[b])

        @pl.when(sp)
        def _():
            g_row(idsA_v, k, b)
            g_eq_wait(b)
            scat(b, dstA, ssemA[b])
            scat_wait(b, ssemA[b])
            g_row(idsB_v, k, b)
            g_eq_wait(b)
            scat(b, dstB, ssemB[b])
            scat_wait(b, ssemB[b])

    for b in range(SLOTS):
        issue(b, b)

    def pipe_body(step, _):
        for b in range(SLOTS):
            k = step * SLOTS + b
            consume(k, b)
            issue(k + SLOTS, b)
        return 0

    lax.fori_loop(0, NSTEPS - 1, pipe_body, 0)

    for b in range(SLOTS):
        consume((NSTEPS - 1) * SLOTS + b, b)


@jax.jit
def _sc_embed(ids_flat, weights):
    mesh = plsc.VectorSubcoreMesh(
        core_axis_name="c", subcore_axis_name="s",
        num_cores=NC, num_subcores=NS)
    f = pl.kernel(
        _sc_body,
        out_type=jax.ShapeDtypeStruct((B * S, D), jnp.float32),
        mesh=mesh,
        compiler_params=pltpu.CompilerParams(needs_layout_passes=False),
        scratch_types=[
            pltpu.VMEM((WTOK,), jnp.int32),
            pltpu.VMEM((WTOK,), jnp.int32),
            pltpu.VMEM((SLOTS * UPAD,), jnp.int32),
            pltpu.VMEM((SLOTS, UNION), jnp.int32),
            pltpu.VMEM((L,), jnp.int32),
            pltpu.VMEM((2 * NS * L,), jnp.int32),
            pltpu.VMEM((SLOTS, UNION, D), jnp.float32),
            pltpu.VMEM_SHARED((2 * NS * L,), jnp.int32),
            pltpu.SemaphoreType.DMA,
            pltpu.SemaphoreType.DMA,
            pltpu.SemaphoreType.DMA,
        ],
    )
    return f(ids_flat, weights)


def kernel(input_ids, weights):
    out = _sc_embed(input_ids.reshape(-1), weights)
    return out.reshape(B, S, D)


# speedup vs baseline: 2.6561x; 1.0053x over previous
import jax
import jax.numpy as jnp
from jax import lax
from jax.experimental import pallas as pl
from jax.experimental.pallas import tpu as pltpu
from jax.experimental.pallas import tpu_sc as plsc

PAD = 1
B = 4
S = 8192
D = 1024

NC = 2
NS = 16
L = 16

WTOK = S // NS
CHUNK = 64
SLACK = 4
UNION = CHUNK + SLACK
NPAIRS = WTOK // CHUNK
SLOTS = 1
NSTEPS = NPAIRS // SLOTS
UPAD = (UNION + L - 1) // L * L


def _sc_body(ids_hbm, w_hbm, out_hbm, idsA_v, idsB_v, uidx_v, sidx_v,
             stage_v, tot_v, rows_v, tot_sh, gA0, sA0, sB0):
    gsemA = [gA0]
    ssemA = [sA0]
    ssemB = [sB0]
    cid = lax.axis_index("c")
    sid = lax.axis_index("s")
    tbaseA = (2 * cid) * S + sid * WTOK
    tbaseB = (2 * cid + 1) * S + sid * WTOK

    pltpu.sync_copy(ids_hbm.at[pl.ds(tbaseA, WTOK)], idsA_v)
    pltpu.sync_copy(ids_hbm.at[pl.ds(tbaseB, WTOK)], idsB_v)

    def make_cs(ref):
        def cs_body(i, carry):
            v = ref[pl.ds(i * L, L)]
            m = jnp.where(v != PAD, 1, 0).astype(jnp.int32)
            c = plsc.cumsum(m) + carry
            ref[pl.ds(i * L, L)] = c * m
            return c[L - 1]
        return cs_body

    totalA = lax.fori_loop(0, WTOK // L, make_cs(idsA_v), jnp.int32(0))
    totalB = lax.fori_loop(0, WTOK // L, make_cs(idsB_v), jnp.int32(0))

    stage_v[...] = jnp.full((L,), totalA, jnp.int32)
    pltpu.sync_copy(stage_v, tot_sh.at[pl.ds(sid * L, L)])
    stage_v[...] = jnp.full((L,), totalB, jnp.int32)
    pltpu.sync_copy(stage_v, tot_sh.at[pl.ds((NS + sid) * L, L)])
    plsc.subcore_barrier()
    pltpu.sync_copy(tot_sh, tot_v)

    offA = jnp.int32(0)
    offB = jnp.int32(0)
    for j in range(NS):
        tA = tot_v[pl.ds(j * L, L)][0]
        tB = tot_v[pl.ds((NS + j) * L, L)][0]
        keep = j < sid
        offA = offA + jnp.where(keep, tA, 0).astype(jnp.int32)
        offB = offB + jnp.where(keep, tB, 0).astype(jnp.int32)

    def make_idx(ref, off):
        def idx_body(i, _):
            e = ref[pl.ds(i * L, L)]
            m = jnp.where(e > 0, 1, 0).astype(jnp.int32)
            ref[pl.ds(i * L, L)] = e + off * m + PAD
            return 0
        return idx_body

    lax.fori_loop(0, WTOK // L, make_idx(idsA_v, offA), 0)
    lax.fori_loop(0, WTOK // L, make_idx(idsB_v, offB), 0)

    lane = lax.broadcasted_iota(jnp.int32, (L,), 0)

    def run_continues(ref, f, d, k):
        lo = jnp.maximum(k * CHUNK - L, 0)
        vprev = ref[pl.ds(lo, L)]
        mism_lo = jnp.sum(jnp.where(
            jnp.logical_and(lane >= L - d, vprev != f - L + lane), 1, 0))
        hi = jnp.minimum(k * CHUNK + CHUNK, WTOK - L)
        vnext = ref[pl.ds(hi, L)]
        mism_hi = jnp.sum(jnp.where(
            jnp.logical_and(lane < SLACK - d, vnext != f + CHUNK + lane),
            1, 0))
        return (mism_lo + mism_hi) == 0

    def flags(k):
        vA = idsA_v[pl.ds(k * CHUNK, L)]
        wA = idsA_v[pl.ds(k * CHUNK + CHUNK - L, L)]
        vB = idsB_v[pl.ds(k * CHUNK, L)]
        wB = idsB_v[pl.ds(k * CHUNK + CHUNK - L, L)]
        fA, lA = vA[0], wA[L - 1]
        fB, lB = vB[0], wB[L - 1]
        cleanA = jnp.logical_and(fA > 1, lA - fA == CHUNK - 1)
        cleanB = jnp.logical_and(fB > 1, lB - fB == CHUNK - 1)
        both = jnp.logical_and(cleanA, cleanB)
        fmin = jnp.minimum(fA, fB)
        dA = fA - fmin
        dB = fB - fmin
        dmax = jnp.maximum(dA, dB)
        eq = jnp.logical_and(both, dmax == 0)
        un = jnp.logical_and(jnp.logical_and(both, dmax > 0),
                             dmax <= SLACK)
        ok = jnp.where(dA > 0, run_continues(idsA_v, fA, dA, k),
                       run_continues(idsB_v, fB, dB, k))
        inner = jnp.logical_and(k >= 1, k < NPAIRS - 1)
        un = jnp.logical_and(un, jnp.logical_and(ok, inner))
        sp = jnp.logical_not(jnp.logical_or(eq, un))
        return eq, un, sp, fmin, dA, dB

    def g_eq(k, b):
        pltpu.async_copy(
            w_hbm.at[idsA_v.at[pl.ds(k * CHUNK, CHUNK)]],
            rows_v.at[b, pl.ds(0, CHUNK)], gsemA[b])

    def g_eq_wait(b):
        pltpu.make_async_copy(
            w_hbm.at[idsA_v.at[pl.ds(0, CHUNK)]],
            rows_v.at[b, pl.ds(0, CHUNK)], gsemA[b]).wait()

    def g_un(fmin, b):
        for g in range(UPAD // L):
            uidx_v[pl.ds(b * UPAD + g * L, L)] = fmin + g * L + lane
        pltpu.async_copy(
            w_hbm.at[uidx_v.at[pl.ds(b * UPAD, UNION)]],
            rows_v.at[b, pl.ds(0, UNION)], gsemA[b])

    def g_un_wait(b):
        pltpu.make_async_copy(
            w_hbm.at[uidx_v.at[pl.ds(0, UNION)]],
            rows_v.at[b, pl.ds(0, UNION)], gsemA[b]).wait()

    def g_row(idx_ref, k, b):
        pltpu.async_copy(
            w_hbm.at[idx_ref.at[pl.ds(k * CHUNK, CHUNK)]],
            rows_v.at[b, pl.ds(0, CHUNK)], gsemA[b])

    def scat(b, dst, sem):
        pltpu.async_copy(
            rows_v.at[b, pl.ds(0, CHUNK)],
            out_hbm.at[pl.ds(dst, CHUNK)], sem)

    def scat_wait(b, sem):
        pltpu.make_async_copy(
            rows_v.at[b, pl.ds(0, CHUNK)], out_hbm.at[pl.ds(0, CHUNK)],
            sem).wait()

    def scat36(b, t0, sem):
        for g in range(UNION // L):
            sidx_v[b, pl.ds(g * L, L)] = t0 + g * L + lane
        sidx_v[b, pl.ds(UNION - L, L)] = t0 + UNION - L + lane
        pltpu.async_copy(
            rows_v.at[b, pl.ds(0, UNION)], out_hbm.at[sidx_v.at[b]], sem)

    def scat36_wait(b, sem):
        pltpu.make_async_copy(
            rows_v.at[b, pl.ds(0, UNION)], out_hbm.at[sidx_v.at[b]],
            sem).wait()

    def issue(k, b):
        eq, un, sp, fmin, dA, dB = flags(k)

        @pl.when(eq)
        def _():
            g_eq(k, b)

        @pl.when(un)
        def _():
            g_un(fmin, b)

    def consume(k, b):
        eq, un, sp, fmin, dA, dB = flags(k)
        dstA = tbaseA + k * CHUNK
        dstB = tbaseB + k * CHUNK

        @pl.when(eq)
        def _():
            g_eq_wait(b)
            scat(b, dstA, ssemA[b])
            scat(b, dstB, ssemB[b])
            scat_wait(b, ssemA[b])
            scat_wait(b, ssemB[b])

        @pl.when(un)
        def _():
            g_un_wait(b)

            @pl.when(dA > 0)
            def _():
                scat36(b, dstA - dA, ssemA[b])
                scat(b, dstB, ssemB[b])
                scat36_wait(b, ssemA[b])
                scat_wait(b, ssemB[b])

            @pl.when(dA == 0)
            def _():
                scat(b, dstA, ssemA[b])
                scat36(b, dstB - dB, ssemB[b])
                scat_wait(b, ssemA[b])
                scat36_wait(b, ssemB[b])

        @pl.when(sp)
        def _():
            g_row(idsA_v, k, b)
            g_eq_wait(b)
            scat(b, dstA, ssemA[b])
            scat_wait(b, ssemA[b])
            g_row(idsB_v, k, b)
            g_eq_wait(b)
            scat(b, dstB, ssemB[b])
            scat_wait(b, ssemB[b])

    for b in range(SLOTS):
        issue(b, b)

    def pipe_body(step, _):
        for b in range(SLOTS):
            k = step * SLOTS + b
            consume(k, b)
            issue(k + SLOTS, b)
        return 0

    lax.fori_loop(0, NSTEPS - 1, pipe_body, 0)

    for b in range(SLOTS):
        consume((NSTEPS - 1) * SLOTS + b, b)


@jax.jit
def _sc_embed(ids_flat, weights):
    mesh = plsc.VectorSubcoreMesh(
        core_axis_name="c", subcore_axis_name="s",
        num_cores=NC, num_subcores=NS)
    f = pl.kernel(
        _sc_body,
        out_type=jax.ShapeDtypeStruct((B * S, D), jnp.float32),
        mesh=mesh,
        compiler_params=pltpu.CompilerParams(needs_layout_passes=False),
        scratch_types=[
            pltpu.VMEM((WTOK,), jnp.int32),
            pltpu.VMEM((WTOK,), jnp.int32),
            pltpu.VMEM((SLOTS * UPAD,), jnp.int32),
            pltpu.VMEM((SLOTS, UNION), jnp.int32),
            pltpu.VMEM((L,), jnp.int32),
            pltpu.VMEM((2 * NS * L,), jnp.int32),
            pltpu.VMEM((SLOTS, UNION, D), jnp.float32),
            pltpu.VMEM_SHARED((2 * NS * L,), jnp.int32),
            pltpu.SemaphoreType.DMA,
            pltpu.SemaphoreType.DMA,
            pltpu.SemaphoreType.DMA,
        ],
    )
    return f(ids_flat, weights)


def kernel(input_ids, weights):
    out = _sc_embed(input_ids.reshape(-1), weights)
    return out.reshape(B, S, D)


# speedup vs baseline: 2.6725x; 1.0062x over previous
import jax
import jax.numpy as jnp
from jax import lax
from jax.experimental import pallas as pl
from jax.experimental.pallas import tpu as pltpu
from jax.experimental.pallas import tpu_sc as plsc

PAD = 1
B = 4
S = 8192
D = 1024

NC = 2
NS = 16
L = 16

WTOK = S // NS
CHUNK = 64
SLACK = 4
UNION = CHUNK + SLACK
NPAIRS = WTOK // CHUNK
SLOTS = 1
NSTEPS = NPAIRS // SLOTS
UPAD = (UNION + L - 1) // L * L


def _sc_body(ids_hbm, w_hbm, out_hbm, idsA_v, idsB_v, uidx_v, sidx_v,
             stage_v, tot_v, rows_v, tot_sh, gA0, sA0, sB0):
    gsemA = [gA0]
    ssemA = [sA0]
    ssemB = [sB0]
    cid = lax.axis_index("c")
    sid = lax.axis_index("s")
    tbaseA = (2 * cid) * S + sid * WTOK
    tbaseB = (2 * cid + 1) * S + sid * WTOK

    pltpu.sync_copy(ids_hbm.at[pl.ds(tbaseA, WTOK)], idsA_v)
    pltpu.sync_copy(ids_hbm.at[pl.ds(tbaseB, WTOK)], idsB_v)

    def cs_body(i, carry):
        ca, cb = carry
        va = idsA_v[pl.ds(i * L, L)]
        vb = idsB_v[pl.ds(i * L, L)]
        ma = jnp.where(va != PAD, 1, 0).astype(jnp.int32)
        mb = jnp.where(vb != PAD, 1, 0).astype(jnp.int32)
        a = plsc.cumsum(ma) + ca
        bb = plsc.cumsum(mb) + cb
        idsA_v[pl.ds(i * L, L)] = a * ma
        idsB_v[pl.ds(i * L, L)] = bb * mb
        return a[L - 1], bb[L - 1]

    totalA, totalB = lax.fori_loop(
        0, WTOK // L, cs_body, (jnp.int32(0), jnp.int32(0)))

    stage_v[...] = jnp.full((L,), totalA, jnp.int32)
    pltpu.sync_copy(stage_v, tot_sh.at[pl.ds(sid * L, L)])
    stage_v[...] = jnp.full((L,), totalB, jnp.int32)
    pltpu.sync_copy(stage_v, tot_sh.at[pl.ds((NS + sid) * L, L)])
    plsc.subcore_barrier()
    pltpu.sync_copy(tot_sh, tot_v)

    offA = jnp.int32(0)
    offB = jnp.int32(0)
    for j in range(NS):
        tA = tot_v[pl.ds(j * L, L)][0]
        tB = tot_v[pl.ds((NS + j) * L, L)][0]
        keep = j < sid
        offA = offA + jnp.where(keep, tA, 0).astype(jnp.int32)
        offB = offB + jnp.where(keep, tB, 0).astype(jnp.int32)

    def idx_body(i, _):
        ea = idsA_v[pl.ds(i * L, L)]
        eb = idsB_v[pl.ds(i * L, L)]
        ma = jnp.where(ea > 0, 1, 0).astype(jnp.int32)
        mb = jnp.where(eb > 0, 1, 0).astype(jnp.int32)
        idsA_v[pl.ds(i * L, L)] = ea + offA * ma + PAD
        idsB_v[pl.ds(i * L, L)] = eb + offB * mb + PAD
        return 0

    lax.fori_loop(0, WTOK // L, idx_body, 0)

    lane = lax.broadcasted_iota(jnp.int32, (L,), 0)

    def run_continues(ref, f, d, k):
        lo = jnp.maximum(k * CHUNK - L, 0)
        vprev = ref[pl.ds(lo, L)]
        mism_lo = jnp.sum(jnp.where(
            jnp.logical_and(lane >= L - d, vprev != f - L + lane), 1, 0))
        hi = jnp.minimum(k * CHUNK + CHUNK, WTOK - L)
        vnext = ref[pl.ds(hi, L)]
        mism_hi = jnp.sum(jnp.where(
            jnp.logical_and(lane < SLACK - d, vnext != f + CHUNK + lane),
            1, 0))
        return (mism_lo + mism_hi) == 0

    def flags(k):
        vA = idsA_v[pl.ds(k * CHUNK, L)]
        wA = idsA_v[pl.ds(k * CHUNK + CHUNK - L, L)]
        vB = idsB_v[pl.ds(k * CHUNK, L)]
        wB = idsB_v[pl.ds(k * CHUNK + CHUNK - L, L)]
        fA, lA = vA[0], wA[L - 1]
        fB, lB = vB[0], wB[L - 1]
        cleanA = jnp.logical_and(fA > 1, lA - fA == CHUNK - 1)
        cleanB = jnp.logical_and(fB > 1, lB - fB == CHUNK - 1)
        both = jnp.logical_and(cleanA, cleanB)
        fmin = jnp.minimum(fA, fB)
        dA = fA - fmin
        dB = fB - fmin
        dmax = jnp.maximum(dA, dB)
        eq = jnp.logical_and(both, dmax == 0)
        un = jnp.logical_and(jnp.logical_and(both, dmax > 0),
                             dmax <= SLACK)
        ok = jnp.where(dA > 0, run_continues(idsA_v, fA, dA, k),
                       run_continues(idsB_v, fB, dB, k))
        inner = jnp.logical_and(k >= 1, k < NPAIRS - 1)
        un = jnp.logical_and(un, jnp.logical_and(ok, inner))
        sp = jnp.logical_not(jnp.logical_or(eq, un))
        return eq, un, sp, fmin, dA, dB

    def g_eq(k, b):
        pltpu.async_copy(
            w_hbm.at[idsA_v.at[pl.ds(k * CHUNK, CHUNK)]],
            rows_v.at[b, pl.ds(0, CHUNK)], gsemA[b])

    def g_eq_wait(b):
        pltpu.make_async_copy(
            w_hbm.at[idsA_v.at[pl.ds(0, CHUNK)]],
            rows_v.at[b, pl.ds(0, CHUNK)], gsemA[b]).wait()

    def g_un(fmin, b):
        for g in range(UPAD // L):
            uidx_v[pl.ds(b * UPAD + g * L, L)] = fmin + g * L + lane
        pltpu.async_copy(
            w_hbm.at[uidx_v.at[pl.ds(b * UPAD, UNION)]],
            rows_v.at[b, pl.ds(0, UNION)], gsemA[b])

    def g_un_wait(b):
        pltpu.make_async_copy(
            w_hbm.at[uidx_v.at[pl.ds(0, UNION)]],
            rows_v.at[b, pl.ds(0, UNION)], gsemA[b]).wait()

    def g_row(idx_ref, k, b):
        pltpu.async_copy(
            w_hbm.at[idx_ref.at[pl.ds(k * CHUNK, CHUNK)]],
            rows_v.at[b, pl.ds(0, CHUNK)], gsemA[b])

    def scat(b, dst, sem):
        pltpu.async_copy(
            rows_v.at[b, pl.ds(0, CHUNK)],
            out_hbm.at[pl.ds(dst, CHUNK)], sem)

    def scat_wait(b, sem):
        pltpu.make_async_copy(
            rows_v.at[b, pl.ds(0, CHUNK)], out_hbm.at[pl.ds(0, CHUNK)],
            sem).wait()

    def scat36(b, t0, sem):
        for g in range(UNION // L):
            sidx_v[b, pl.ds(g * L, L)] = t0 + g * L + lane
        sidx_v[b, pl.ds(UNION - L, L)] = t0 + UNION - L + lane
        pltpu.async_copy(
            rows_v.at[b, pl.ds(0, UNION)], out_hbm.at[sidx_v.at[b]], sem)

    def scat36_wait(b, sem):
        pltpu.make_async_copy(
            rows_v.at[b, pl.ds(0, UNION)], out_hbm.at[sidx_v.at[b]],
            sem).wait()

    def issue(k, b):
        eq, un, sp, fmin, dA, dB = flags(k)

        @pl.when(eq)
        def _():
            g_eq(k, b)

        @pl.when(un)
        def _():
            g_un(fmin, b)

    def consume(k, b):
        eq, un, sp, fmin, dA, dB = flags(k)
        dstA = tbaseA + k * CHUNK
        dstB = tbaseB + k * CHUNK

        @pl.when(eq)
        def _():
            g_eq_wait(b)
            scat(b, dstA, ssemA[b])
            scat(b, dstB, ssemB[b])
            scat_wait(b, ssemA[b])
            scat_wait(b, ssemB[b])

        @pl.when(un)
        def _():
            g_un_wait(b)

            @pl.when(dA > 0)
            def _():
                scat36(b, dstA - dA, ssemA[b])
                scat(b, dstB, ssemB[b])
                scat36_wait(b, ssemA[b])
                scat_wait(b, ssemB[b])

            @pl.when(dA == 0)
            def _():
                scat(b, dstA, ssemA[b])
                scat36(b, dstB - dB, ssemB[b])
                scat_wait(b, ssemA[b])
                scat36_wait(b, ssemB[b])

        @pl.when(sp)
        def _():
            g_row(idsA_v, k, b)
            g_eq_wait(b)
            scat(b, dstA, ssemA[b])
            scat_wait(b, ssemA[b])
            g_row(idsB_v, k, b)
            g_eq_wait(b)
            scat(b, dstB, ssemB[b])
            scat_wait(b, ssemB[b])

    for b in range(SLOTS):
        issue(b, b)

    def pipe_body(step, _):
        for b in range(SLOTS):
            k = step * SLOTS + b
            consume(k, b)
            issue(k + SLOTS, b)
        return 0

    lax.fori_loop(0, NSTEPS - 1, pipe_body, 0)

    for b in range(SLOTS):
        consume((NSTEPS - 1) * SLOTS + b, b)


@jax.jit
def _sc_embed(ids_flat, weights):
    mesh = plsc.VectorSubcoreMesh(
        core_axis_name="c", subcore_axis_name="s",
        num_cores=NC, num_subcores=NS)
    f = pl.kernel(
        _sc_body,
        out_type=jax.ShapeDtypeStruct((B * S, D), jnp.float32),
        mesh=mesh,
        compiler_params=pltpu.CompilerParams(needs_layout_passes=False),
        scratch_types=[
            pltpu.VMEM((WTOK,), jnp.int32),
            pltpu.VMEM((WTOK,), jnp.int32),
            pltpu.VMEM((SLOTS * UPAD,), jnp.int32),
            pltpu.VMEM((SLOTS, UNION), jnp.int32),
            pltpu.VMEM((L,), jnp.int32),
            pltpu.VMEM((2 * NS * L,), jnp.int32),
            pltpu.VMEM((SLOTS, UNION, D), jnp.float32),
            pltpu.VMEM_SHARED((2 * NS * L,), jnp.int32),
            pltpu.SemaphoreType.DMA,
            pltpu.SemaphoreType.DMA,
            pltpu.SemaphoreType.DMA,
        ],
    )
    return f(ids_flat, weights)


def kernel(input_ids, weights):
    out = _sc_embed(input_ids.reshape(-1), weights)
    return out.reshape(B, S, D)
